# Initial kernel scaffold; baseline (speedup 1.0000x reference)
import jax
import jax.numpy as jnp
from jax.experimental import pallas as pl


def kernel(x, edge_index, W1, b1, W2, b2, Wr, br):
    raise NotImplementedError("write your pallas kernel here")


# speedup vs baseline: 22.9888x; 22.9888x over previous
import functools

import jax
import jax.numpy as jnp
from jax import lax
from jax.experimental import pallas as pl
from jax.experimental.pallas import tpu as pltpu
from jax.experimental.pallas import tpu_sc as plsc

N = 10000
E = 320000
D = 128
NC = 2
NS = 16
NW = NC * NS
EP = E // NW
C = 80
J = EP // C
RPT = N // NS
RB = 2000
G = N // RB

_mesh = plsc.VectorSubcoreMesh(core_axis_name="c", subcore_axis_name="s")
_sc_params = pltpu.CompilerParams(
    use_tc_tiling_on_sc=False, needs_layout_passes=False)


def _tc_xw_body(x_ref, w1_ref, xw_ref):
    xw_ref[...] = jnp.dot(x_ref[...], w1_ref[...],
                          precision=lax.Precision.HIGHEST)


_tc_xw = pl.pallas_call(
    _tc_xw_body,
    grid=(G,),
    in_specs=[
        pl.BlockSpec((RB, D), lambda i: (i, 0)),
        pl.BlockSpec((D, D), lambda i: (0, 0)),
    ],
    out_specs=[pl.BlockSpec((RB, D), lambda i: (i, 0))],
    out_shape=[jax.ShapeDtypeStruct((N, D), jnp.float32)],
)


@functools.partial(
    pl.kernel,
    out_type=jax.ShapeDtypeStruct((NC, 2, N), jnp.float32),
    mesh=_mesh,
    compiler_params=_sc_params,
    scratch_types=[
        pltpu.VMEM((J, C), jnp.int32),
        pltpu.VMEM((J, C), jnp.int32),
        pltpu.VMEM((2, N), jnp.float32),
        pltpu.VMEM((2,), jnp.int32),
        pltpu.VMEM_SHARED((2, N), jnp.float32),
    ],
)
def _sc_degrees(src_hbm, dst_hbm, i01_hbm, out_hbm, sidx_v, didx_v, deg_v, i01_v, hsum_sh):
    cid = lax.axis_index("c")
    sid = lax.axis_index("s")
    w = sid * NC + cid
    pltpu.sync_copy(src_hbm.at[w], sidx_v)
    pltpu.sync_copy(dst_hbm.at[w], didx_v)
    pltpu.sync_copy(i01_hbm, i01_v)
    z16 = jnp.zeros((16,), jnp.float32)

    def zbody(i, carry):
        deg_v[0, pl.ds(i * 16, 16)] = z16
        deg_v[1, pl.ds(i * 16, 16)] = z16
        return carry

    lax.fori_loop(0, N // 16, zbody, 0, unroll=False)

    @pl.when(sid == 0)
    def _():
        pltpu.sync_copy(deg_v, hsum_sh)

    ones = jnp.ones((16,), jnp.float32)

    def body(j, carry):
        for k in range(C // 16):
            s16 = sidx_v[j, pl.ds(k * 16, 16)]
            plsc.addupdate_scatter(deg_v.at[0], [s16], ones)
            d16 = didx_v[j, pl.ds(k * 16, 16)]
            plsc.addupdate_scatter(deg_v.at[1], [d16], ones)
        return carry

    lax.fori_loop(0, J, body, 0, unroll=False)
    plsc.subcore_barrier()
    pltpu.sync_copy(deg_v, hsum_sh.at[i01_v], add=True)
    plsc.subcore_barrier()

    @pl.when(sid == 0)
    def _():
        pltpu.sync_copy(hsum_sh, out_hbm.at[cid])


def _tc_prep_body(hp_ref, xw_ref, y_ref, ns_ref, nd_ref):
    deg = jnp.sum(hp_ref[...], axis=-1) + 1.0
    ns = lax.rsqrt(deg[0])
    nd = lax.rsqrt(deg[1])
    y_ref[...] = xw_ref[...] * ns[:, None]
    ns_ref[...] = ns[:, None]
    nd_ref[...] = nd[:, None]


_tc_prep = pl.pallas_call(
    _tc_prep_body,
    grid=(G,),
    in_specs=[
        pl.BlockSpec((2, RB, NC), lambda i: (0, i, 0)),
        pl.BlockSpec((RB, D), lambda i: (i, 0)),
    ],
    out_specs=[
        pl.BlockSpec((RB, D), lambda i: (i, 0)),
        pl.BlockSpec((RB, 1), lambda i: (i, 0)),
        pl.BlockSpec((RB, 1), lambda i: (i, 0)),
    ],
    out_shape=[
        jax.ShapeDtypeStruct((N, D), jnp.float32),
        jax.ShapeDtypeStruct((N, 1), jnp.float32),
        jax.ShapeDtypeStruct((N, 1), jnp.float32),
    ],
)


@functools.partial(
    pl.kernel,
    out_type=jax.ShapeDtypeStruct((NC, N, D), jnp.float32),
    mesh=_mesh,
    compiler_params=_sc_params,
    scratch_types=[
        pltpu.VMEM((J, C), jnp.int32),
        pltpu.VMEM((J, C), jnp.int32),
        pltpu.VMEM((C, D), jnp.float32),
        pltpu.VMEM((C, D), jnp.float32),
        pltpu.VMEM_SHARED((N, D), jnp.float32),
        pltpu.SemaphoreType.DMA,
        pltpu.SemaphoreType.DMA,
    ],
)
def _sc_gather_scatter(src_hbm, dst_hbm, y_hbm, zrows_hbm, out_hbm,
                       sidx_v, didx_v, rows0_v, rows1_v, acc_sh, sem0, sem1):
    cid = lax.axis_index("c")
    sid = lax.axis_index("s")
    w = sid * NC + cid

    @pl.when(cid == 0)
    def _():
        pltpu.sync_copy(y_hbm.at[pl.ds(sid * RPT, RPT)],
                        acc_sh.at[pl.ds(sid * RPT, RPT)])

    @pl.when(cid != 0)
    def _():
        pltpu.sync_copy(zrows_hbm.at[pl.ds(sid * RPT, RPT)],
                        acc_sh.at[pl.ds(sid * RPT, RPT)])

    pltpu.sync_copy(src_hbm.at[w], sidx_v)
    pltpu.sync_copy(dst_hbm.at[w], didx_v)
    plsc.subcore_barrier()

    pltpu.async_copy(y_hbm.at[sidx_v.at[0]], rows0_v, sem0)

    def _drain(buf, sem):
        pltpu.make_async_copy(y_hbm.at[sidx_v.at[0]], buf, sem).wait()

    def body(i, carry):
        j = 2 * i
        pltpu.async_copy(y_hbm.at[sidx_v.at[j + 1]], rows1_v, sem1)
        _drain(rows0_v, sem0)
        pltpu.sync_copy(rows0_v, acc_sh.at[didx_v.at[j]], add=True)
        pltpu.async_copy(y_hbm.at[sidx_v.at[j + 2]], rows0_v, sem0)
        _drain(rows1_v, sem1)
        pltpu.sync_copy(rows1_v, acc_sh.at[didx_v.at[j + 1]], add=True)
        return carry

    lax.fori_loop(0, (J - 1) // 2, body, 0, unroll=False)
    _drain(rows0_v, sem0)
    pltpu.sync_copy(rows0_v, acc_sh.at[didx_v.at[J - 1]], add=True)
    plsc.subcore_barrier()
    pltpu.sync_copy(acc_sh.at[pl.ds(sid * RPT, RPT)],
                    out_hbm.at[cid].at[pl.ds(sid * RPT, RPT)])


def _tc_mid_body(acc_ref, ns_ref, nd_ref, b1_ref, w2_ref, wr_ref, t_ref):
    w = jnp.dot(w2_ref[...], wr_ref[...],
                precision=lax.Precision.HIGHEST)
    a = acc_ref[0] + acc_ref[1]
    h1 = jnp.maximum(nd_ref[...] * a + b1_ref[...][None, :], 0.0)
    t_ref[...] = jnp.dot(h1, w, precision=lax.Precision.HIGHEST) * ns_ref[...]


_tc_mid = pl.pallas_call(
    _tc_mid_body,
    grid=(G,),
    in_specs=[
        pl.BlockSpec((NC, RB, D), lambda i: (0, i, 0)),
        pl.BlockSpec((RB, 1), lambda i: (i, 0)),
        pl.BlockSpec((RB, 1), lambda i: (i, 0)),
        pl.BlockSpec((D,), lambda i: (0,)),
        pl.BlockSpec((D, D), lambda i: (0, 0)),
        pl.BlockSpec((D, 1), lambda i: (0, 0)),
    ],
    out_specs=[pl.BlockSpec((RB, 1), lambda i: (i, 0))],
    out_shape=[jax.ShapeDtypeStruct((N, 1), jnp.float32)],
)


@functools.partial(
    pl.kernel,
    out_type=jax.ShapeDtypeStruct((NC, N), jnp.float32),
    mesh=_mesh,
    compiler_params=_sc_params,
    scratch_types=[
        pltpu.VMEM((J, C), jnp.int32),
        pltpu.VMEM((J, C), jnp.int32),
        pltpu.VMEM((N,), jnp.float32),
        pltpu.VMEM((1, N), jnp.float32),
        pltpu.VMEM((1,), jnp.int32),
        pltpu.VMEM_SHARED((1, N), jnp.float32),
    ],
)
def _sc_scalar(src_hbm, dst_hbm, t_hbm, i0_hbm, out_hbm,
               sidx_v, didx_v, t_v, z_v, i0_v, zsum_sh):
    cid = lax.axis_index("c")
    sid = lax.axis_index("s")
    w = sid * NC + cid
    pltpu.sync_copy(src_hbm.at[w], sidx_v)
    pltpu.sync_copy(dst_hbm.at[w], didx_v)
    pltpu.sync_copy(t_hbm, t_v)
    pltpu.sync_copy(i0_hbm, i0_v)
    z16 = jnp.zeros((16,), jnp.float32)

    def zbody(i, carry):
        z_v[0, pl.ds(i * 16, 16)] = z16
        return carry

    lax.fori_loop(0, N // 16, zbody, 0, unroll=False)

    @pl.when(sid == 0)
    def _():
        pltpu.sync_copy(z_v, zsum_sh)

    def body(j, carry):
        for k in range(C // 16):
            s16 = sidx_v[j, pl.ds(k * 16, 16)]
            vals = plsc.load_gather(t_v, [s16])
            d16 = didx_v[j, pl.ds(k * 16, 16)]
            plsc.addupdate_scatter(z_v.at[0], [d16], vals)
        return carry

    lax.fori_loop(0, J, body, 0, unroll=False)
    plsc.subcore_barrier()
    pltpu.sync_copy(z_v, zsum_sh.at[i0_v], add=True)
    plsc.subcore_barrier()

    @pl.when(sid == 0)
    def _():
        pltpu.sync_copy(zsum_sh.at[0], out_hbm.at[cid])


def _tc_final_body(zp_ref, t_ref, nd_ref, b2_ref, wr_ref, br_ref, out_ref):
    z = jnp.sum(zp_ref[...], axis=-1, keepdims=True)
    c = jnp.sum(b2_ref[...] * wr_ref[...][:, 0]) + br_ref[0]
    out_ref[...] = nd_ref[...] * (z + t_ref[...]) + c


_tc_final = pl.pallas_call(
    _tc_final_body,
    grid=(G,),
    in_specs=[
        pl.BlockSpec((RB, NC), lambda i: (i, 0)),
        pl.BlockSpec((RB, 1), lambda i: (i, 0)),
        pl.BlockSpec((RB, 1), lambda i: (i, 0)),
        pl.BlockSpec((D,), lambda i: (0,)),
        pl.BlockSpec((D, 1), lambda i: (0, 0)),
        pl.BlockSpec((1,), lambda i: (0,)),
    ],
    out_specs=[pl.BlockSpec((RB, 1), lambda i: (i, 0))],
    out_shape=[jax.ShapeDtypeStruct((N, 1), jnp.float32)],
)


def kernel(x, edge_index, W1, b1, W2, b2, Wr, br):
    src = edge_index[0].reshape(NW, J, C)
    dst = edge_index[1].reshape(NW, J, C)
    zrows = jnp.zeros((N, D), jnp.float32)
    i01 = jnp.arange(2, dtype=jnp.int32)
    i0 = jnp.zeros((1,), jnp.int32)

    (xw,) = _tc_xw(x, W1)
    hp = _sc_degrees(src, dst, i01)
    hp_t = jnp.transpose(hp, (1, 2, 0))
    y, ns, nd = _tc_prep(hp_t, xw)
    acc = _sc_gather_scatter(src, dst, y, zrows)
    (t,) = _tc_mid(acc, ns, nd, b1, W2, Wr)
    zp = _sc_scalar(src, dst, t[:, 0], i0)
    zp_t = jnp.transpose(zp, (1, 0))
    (out,) = _tc_final(zp_t, t, nd, b2, Wr, br)
    return out[:, 0]


# speedup vs baseline: 24.8344x; 1.0803x over previous
import functools

import jax
import jax.numpy as jnp
from jax import lax
from jax.experimental import pallas as pl
from jax.experimental.pallas import tpu as pltpu
from jax.experimental.pallas import tpu_sc as plsc

N = 10000
E = 320000
D = 128
NC = 2
NS = 16
NW = NC * NS
EP = E // NW
C = 80
J = EP // C
RPT = N // NS
RB = 2000
G = N // RB

_mesh = plsc.VectorSubcoreMesh(core_axis_name="c", subcore_axis_name="s")
_sc_params = pltpu.CompilerParams(
    use_tc_tiling_on_sc=False, needs_layout_passes=False)


@functools.partial(
    pl.kernel,
    out_type=jax.ShapeDtypeStruct((NC, 2, N), jnp.float32),
    mesh=_mesh,
    compiler_params=_sc_params,
    scratch_types=[
        pltpu.VMEM((J, C), jnp.int32),
        pltpu.VMEM((J, C), jnp.int32),
        pltpu.VMEM((2, N), jnp.float32),
        pltpu.VMEM((2,), jnp.int32),
        pltpu.VMEM_SHARED((2, N), jnp.float32),
    ],
)
def _sc_degrees(src_hbm, dst_hbm, i01_hbm, out_hbm, sidx_v, didx_v, deg_v, i01_v, hsum_sh):
    cid = lax.axis_index("c")
    sid = lax.axis_index("s")
    w = sid * NC + cid
    pltpu.sync_copy(src_hbm.at[w], sidx_v)
    pltpu.sync_copy(dst_hbm.at[w], didx_v)
    pltpu.sync_copy(i01_hbm, i01_v)
    z16 = jnp.zeros((16,), jnp.float32)

    def zbody(i, carry):
        deg_v[0, pl.ds(i * 16, 16)] = z16
        deg_v[1, pl.ds(i * 16, 16)] = z16
        return carry

    lax.fori_loop(0, N // 16, zbody, 0, unroll=False)

    @pl.when(sid == 0)
    def _():
        pltpu.sync_copy(deg_v, hsum_sh)

    ones = jnp.ones((16,), jnp.float32)

    def body(j, carry):
        for k in range(C // 16):
            s16 = sidx_v[j, pl.ds(k * 16, 16)]
            plsc.addupdate_scatter(deg_v.at[0], [s16], ones)
            d16 = didx_v[j, pl.ds(k * 16, 16)]
            plsc.addupdate_scatter(deg_v.at[1], [d16], ones)
        return carry

    lax.fori_loop(0, J, body, 0, unroll=False)
    plsc.subcore_barrier()
    pltpu.sync_copy(deg_v, hsum_sh.at[i01_v], add=True)
    plsc.subcore_barrier()

    @pl.when(sid == 0)
    def _():
        pltpu.sync_copy(hsum_sh, out_hbm.at[cid])


def _tc_prep_body(hp_ref, x_ref, w1_ref, y_ref, ns_ref, nd_ref):
    deg = jnp.sum(hp_ref[...], axis=-1) + 1.0
    ns = lax.rsqrt(deg[0])
    nd = lax.rsqrt(deg[1])
    y_ref[...] = jnp.dot(x_ref[...] * ns[:, None], w1_ref[...],
                         precision=lax.Precision.HIGHEST)
    ns_ref[...] = ns[:, None]
    nd_ref[...] = nd[:, None]


_tc_prep = pl.pallas_call(
    _tc_prep_body,
    grid=(G,),
    in_specs=[
        pl.BlockSpec((2, RB, NC), lambda i: (0, i, 0)),
        pl.BlockSpec((RB, D), lambda i: (i, 0)),
        pl.BlockSpec((D, D), lambda i: (0, 0)),
    ],
    out_specs=[
        pl.BlockSpec((RB, D), lambda i: (i, 0)),
        pl.BlockSpec((RB, 1), lambda i: (i, 0)),
        pl.BlockSpec((RB, 1), lambda i: (i, 0)),
    ],
    out_shape=[
        jax.ShapeDtypeStruct((N, D), jnp.float32),
        jax.ShapeDtypeStruct((N, 1), jnp.float32),
        jax.ShapeDtypeStruct((N, 1), jnp.float32),
    ],
)


@functools.partial(
    pl.kernel,
    out_type=jax.ShapeDtypeStruct((NC, N, D), jnp.float32),
    mesh=_mesh,
    compiler_params=_sc_params,
    scratch_types=[
        pltpu.VMEM((J, C), jnp.int32),
        pltpu.VMEM((J, C), jnp.int32),
        pltpu.VMEM((C, D), jnp.float32),
        pltpu.VMEM((C, D), jnp.float32),
        pltpu.VMEM((C, D), jnp.float32),
        pltpu.VMEM_SHARED((N, D), jnp.float32),
        pltpu.SemaphoreType.DMA,
        pltpu.SemaphoreType.DMA,
        pltpu.SemaphoreType.DMA,
        pltpu.SemaphoreType.DMA,
        pltpu.SemaphoreType.DMA,
        pltpu.SemaphoreType.DMA,
    ],
)
def _sc_gather_scatter(src_hbm, dst_hbm, y_hbm, zrows_hbm, out_hbm,
                       sidx_v, didx_v, rows0_v, rows1_v, rows2_v,
                       acc_sh, g0, g1, g2, s0, s1, s2):
    cid = lax.axis_index("c")
    sid = lax.axis_index("s")
    w = sid * NC + cid

    @pl.when(cid == 0)
    def _():
        pltpu.sync_copy(y_hbm.at[pl.ds(sid * RPT, RPT)],
                        acc_sh.at[pl.ds(sid * RPT, RPT)])

    @pl.when(cid != 0)
    def _():
        pltpu.sync_copy(zrows_hbm.at[pl.ds(sid * RPT, RPT)],
                        acc_sh.at[pl.ds(sid * RPT, RPT)])

    pltpu.sync_copy(src_hbm.at[w], sidx_v)
    pltpu.sync_copy(dst_hbm.at[w], didx_v)
    plsc.subcore_barrier()

    bufs = (rows0_v, rows1_v, rows2_v)
    gs = (g0, g1, g2)
    ss = (s0, s1, s2)

    def _drain(buf, sem):
        pltpu.make_async_copy(y_hbm.at[sidx_v.at[0]], buf, sem).wait()

    def _gather(j, b):
        pltpu.async_copy(y_hbm.at[sidx_v.at[j]], bufs[b], gs[b])

    def _step(j, b, drain_prev):
        _drain(bufs[b], gs[b])
        pltpu.async_copy(bufs[b], acc_sh.at[didx_v.at[j]], ss[b], add=True)
        bp = (b + 2) % 3
        if drain_prev:
            _drain(bufs[bp], ss[bp])
        return bp

    _gather(0, 0)
    _gather(1, 1)
    _step(0, 0, False)
    _gather(2, 2)
    bp = _step(1, 1, True)
    _gather(3, bp)
    bp = _step(2, 2, True)
    _gather(4, bp)

    def body(i, carry):
        j0 = 3 * i
        for b in range(3):
            bp = _step(j0 + b, b, True)
            _gather(j0 + b + 2, bp)
        return carry

    lax.fori_loop(1, (J - 2) // 3, body, 0, unroll=False)
    _step(J - 2, (J - 2) % 3, True)
    _step(J - 1, (J - 1) % 3, True)
    _drain(bufs[(J - 1) % 3], ss[(J - 1) % 3])
    plsc.subcore_barrier()
    pltpu.sync_copy(acc_sh.at[pl.ds(sid * RPT, RPT)],
                    out_hbm.at[cid].at[pl.ds(sid * RPT, RPT)])


def _tc_mid_body(acc_ref, ns_ref, nd_ref, b1_ref, w2_ref, wr_ref, t_ref):
    w = jnp.dot(w2_ref[...], wr_ref[...],
                precision=lax.Precision.HIGHEST)
    a = acc_ref[0] + acc_ref[1]
    h1 = jnp.maximum(nd_ref[...] * a + b1_ref[...][None, :], 0.0)
    t_ref[...] = jnp.dot(h1, w, precision=lax.Precision.HIGHEST) * ns_ref[...]


_tc_mid = pl.pallas_call(
    _tc_mid_body,
    grid=(G,),
    in_specs=[
        pl.BlockSpec((NC, RB, D), lambda i: (0, i, 0)),
        pl.BlockSpec((RB, 1), lambda i: (i, 0)),
        pl.BlockSpec((RB, 1), lambda i: (i, 0)),
        pl.BlockSpec((D,), lambda i: (0,)),
        pl.BlockSpec((D, D), lambda i: (0, 0)),
        pl.BlockSpec((D, 1), lambda i: (0, 0)),
    ],
    out_specs=[pl.BlockSpec((RB, 1), lambda i: (i, 0))],
    out_shape=[jax.ShapeDtypeStruct((N, 1), jnp.float32)],
)


@functools.partial(
    pl.kernel,
    out_type=jax.ShapeDtypeStruct((NC, N), jnp.float32),
    mesh=_mesh,
    compiler_params=_sc_params,
    scratch_types=[
        pltpu.VMEM((J, C), jnp.int32),
        pltpu.VMEM((J, C), jnp.int32),
        pltpu.VMEM((N,), jnp.float32),
        pltpu.VMEM((1, N), jnp.float32),
        pltpu.VMEM((1,), jnp.int32),
        pltpu.VMEM_SHARED((1, N), jnp.float32),
    ],
)
def _sc_scalar(src_hbm, dst_hbm, t_hbm, i0_hbm, out_hbm,
               sidx_v, didx_v, t_v, z_v, i0_v, zsum_sh):
    cid = lax.axis_index("c")
    sid = lax.axis_index("s")
    w = sid * NC + cid
    pltpu.sync_copy(src_hbm.at[w], sidx_v)
    pltpu.sync_copy(dst_hbm.at[w], didx_v)
    pltpu.sync_copy(t_hbm, t_v)
    pltpu.sync_copy(i0_hbm, i0_v)
    z16 = jnp.zeros((16,), jnp.float32)

    def zbody(i, carry):
        z_v[0, pl.ds(i * 16, 16)] = z16
        return carry

    lax.fori_loop(0, N // 16, zbody, 0, unroll=False)

    @pl.when(sid == 0)
    def _():
        pltpu.sync_copy(z_v, zsum_sh)

    def body(j, carry):
        for k in range(C // 16):
            s16 = sidx_v[j, pl.ds(k * 16, 16)]
            vals = plsc.load_gather(t_v, [s16])
            d16 = didx_v[j, pl.ds(k * 16, 16)]
            plsc.addupdate_scatter(z_v.at[0], [d16], vals)
        return carry

    lax.fori_loop(0, J, body, 0, unroll=False)
    plsc.subcore_barrier()
    pltpu.sync_copy(z_v, zsum_sh.at[i0_v], add=True)
    plsc.subcore_barrier()

    @pl.when(sid == 0)
    def _():
        pltpu.sync_copy(zsum_sh.at[0], out_hbm.at[cid])


def _tc_final_body(zp_ref, t_ref, nd_ref, b2_ref, wr_ref, br_ref, out_ref):
    z = jnp.sum(zp_ref[...], axis=-1, keepdims=True)
    c = jnp.sum(b2_ref[...] * wr_ref[...][:, 0]) + br_ref[0]
    out_ref[...] = nd_ref[...] * (z + t_ref[...]) + c


_tc_final = pl.pallas_call(
    _tc_final_body,
    grid=(G,),
    in_specs=[
        pl.BlockSpec((RB, NC), lambda i: (i, 0)),
        pl.BlockSpec((RB, 1), lambda i: (i, 0)),
        pl.BlockSpec((RB, 1), lambda i: (i, 0)),
        pl.BlockSpec((D,), lambda i: (0,)),
        pl.BlockSpec((D, 1), lambda i: (0, 0)),
        pl.BlockSpec((1,), lambda i: (0,)),
    ],
    out_specs=[pl.BlockSpec((RB, 1), lambda i: (i, 0))],
    out_shape=[jax.ShapeDtypeStruct((N, 1), jnp.float32)],
)


def kernel(x, edge_index, W1, b1, W2, b2, Wr, br):
    src = edge_index[0].reshape(NW, J, C)
    dst = edge_index[1].reshape(NW, J, C)
    zrows = jnp.zeros((N, D), jnp.float32)
    i01 = jnp.arange(2, dtype=jnp.int32)
    i0 = jnp.zeros((1,), jnp.int32)

    hp = _sc_degrees(src, dst, i01)
    hp_t = jnp.transpose(hp, (1, 2, 0))
    y, ns, nd = _tc_prep(hp_t, x, W1)
    acc = _sc_gather_scatter(src, dst, y, zrows)
    (t,) = _tc_mid(acc, ns, nd, b1, W2, Wr)
    zp = _sc_scalar(src, dst, t[:, 0], i0)
    zp_t = jnp.transpose(zp, (1, 0))
    (out,) = _tc_final(zp_t, t, nd, b2, Wr, br)
    return out[:, 0]


# speedup vs baseline: 26.9894x; 1.0868x over previous
import functools

import jax
import jax.numpy as jnp
from jax import lax
from jax.experimental import pallas as pl
from jax.experimental.pallas import tpu as pltpu
from jax.experimental.pallas import tpu_sc as plsc

N = 10000
E = 320000
D = 128
NC = 2
NS = 16
NW = NC * NS
EP = E // NW
C = 80
J = EP // C
RPT = N // NS
RB = 2000
G = N // RB

_mesh = plsc.VectorSubcoreMesh(core_axis_name="c", subcore_axis_name="s")
_sc_params = pltpu.CompilerParams(
    use_tc_tiling_on_sc=False, needs_layout_passes=False)


@functools.partial(
    pl.kernel,
    out_type=jax.ShapeDtypeStruct((NC, 2, N), jnp.float32),
    mesh=_mesh,
    compiler_params=_sc_params,
    scratch_types=[
        pltpu.VMEM((J, C), jnp.int32),
        pltpu.VMEM((J, C), jnp.int32),
        pltpu.VMEM((2, N), jnp.float32),
        pltpu.VMEM((2,), jnp.int32),
        pltpu.VMEM_SHARED((2, N), jnp.float32),
    ],
)
def _sc_degrees(edges_hbm, i01_hbm, out_hbm, sidx_v, didx_v, deg_v, i01_v, hsum_sh):
    cid = lax.axis_index("c")
    sid = lax.axis_index("s")
    w = sid * NC + cid
    pltpu.sync_copy(edges_hbm.at[0, w], sidx_v)
    pltpu.sync_copy(edges_hbm.at[1, w], didx_v)
    pltpu.sync_copy(i01_hbm, i01_v)
    z16 = jnp.zeros((16,), jnp.float32)

    def zbody(i, carry):
        deg_v[0, pl.ds(i * 16, 16)] = z16
        deg_v[1, pl.ds(i * 16, 16)] = z16
        return carry

    lax.fori_loop(0, N // 16, zbody, 0, unroll=False)

    @pl.when(sid == 0)
    def _():
        pltpu.sync_copy(deg_v, hsum_sh)

    ones = jnp.ones((16,), jnp.float32)

    def body(j, carry):
        for k in range(C // 16):
            s16 = sidx_v[j, pl.ds(k * 16, 16)]
            plsc.addupdate_scatter(deg_v.at[0], [s16], ones)
            d16 = didx_v[j, pl.ds(k * 16, 16)]
            plsc.addupdate_scatter(deg_v.at[1], [d16], ones)
        return carry

    lax.fori_loop(0, J, body, 0, unroll=False)
    plsc.subcore_barrier()
    pltpu.sync_copy(deg_v, hsum_sh.at[i01_v], add=True)
    plsc.subcore_barrier()

    @pl.when(sid == 0)
    def _():
        pltpu.sync_copy(hsum_sh, out_hbm.at[cid])


def _tc_norms_body(hp_ref, ns_ref, nd_ref):
    deg = hp_ref[0] + hp_ref[1] + 1.0
    ns_ref[...] = lax.rsqrt(deg[0])[:, None]
    nd_ref[...] = lax.rsqrt(deg[1])[:, None]


_tc_norms = pl.pallas_call(
    _tc_norms_body,
    out_shape=[
        jax.ShapeDtypeStruct((N, 1), jnp.float32),
        jax.ShapeDtypeStruct((N, 1), jnp.float32),
    ],
)


def _tc_prep_body(ns_ref, x_ref, w1_ref, y_ref):
    y_ref[...] = jnp.dot(x_ref[...] * ns_ref[...], w1_ref[...],
                         precision=lax.Precision.HIGHEST)


_tc_prep = pl.pallas_call(
    _tc_prep_body,
    grid=(G,),
    in_specs=[
        pl.BlockSpec((RB, 1), lambda i: (i, 0)),
        pl.BlockSpec((RB, D), lambda i: (i, 0)),
        pl.BlockSpec((D, D), lambda i: (0, 0)),
    ],
    out_specs=[pl.BlockSpec((RB, D), lambda i: (i, 0))],
    out_shape=[jax.ShapeDtypeStruct((N, D), jnp.float32)],
)


@functools.partial(
    pl.kernel,
    out_type=jax.ShapeDtypeStruct((NC, N, D), jnp.float32),
    mesh=_mesh,
    compiler_params=_sc_params,
    scratch_types=[
        pltpu.VMEM((J, C), jnp.int32),
        pltpu.VMEM((J, C), jnp.int32),
        pltpu.VMEM((C, D), jnp.float32),
        pltpu.VMEM((C, D), jnp.float32),
        pltpu.VMEM((C, D), jnp.float32),
        pltpu.VMEM_SHARED((N, D), jnp.float32),
        pltpu.SemaphoreType.DMA,
        pltpu.SemaphoreType.DMA,
        pltpu.SemaphoreType.DMA,
        pltpu.SemaphoreType.DMA,
        pltpu.SemaphoreType.DMA,
        pltpu.SemaphoreType.DMA,
    ],
)
def _sc_gather_scatter(edges_hbm, y_hbm, zrows_hbm, out_hbm,
                       sidx_v, didx_v, rows0_v, rows1_v, rows2_v,
                       acc_sh, g0, g1, g2, s0, s1, s2):
    cid = lax.axis_index("c")
    sid = lax.axis_index("s")
    w = sid * NC + cid

    @pl.when(cid == 0)
    def _():
        pltpu.sync_copy(y_hbm.at[pl.ds(sid * RPT, RPT)],
                        acc_sh.at[pl.ds(sid * RPT, RPT)])

    @pl.when(cid != 0)
    def _():
        pltpu.sync_copy(zrows_hbm.at[pl.ds(sid * RPT, RPT)],
                        acc_sh.at[pl.ds(sid * RPT, RPT)])

    pltpu.sync_copy(edges_hbm.at[0, w], sidx_v)
    pltpu.sync_copy(edges_hbm.at[1, w], didx_v)
    plsc.subcore_barrier()

    bufs = (rows0_v, rows1_v, rows2_v)
    gs = (g0, g1, g2)
    ss = (s0, s1, s2)

    def _drain(buf, sem):
        pltpu.make_async_copy(y_hbm.at[sidx_v.at[0]], buf, sem).wait()

    def _gather(j, b):
        pltpu.async_copy(y_hbm.at[sidx_v.at[j]], bufs[b], gs[b])

    def _step(j, b, drain_prev):
        _drain(bufs[b], gs[b])
        pltpu.async_copy(bufs[b], acc_sh.at[didx_v.at[j]], ss[b], add=True)
        bp = (b + 2) % 3
        if drain_prev:
            _drain(bufs[bp], ss[bp])
        return bp

    _gather(0, 0)
    _gather(1, 1)
    _step(0, 0, False)
    _gather(2, 2)
    bp = _step(1, 1, True)
    _gather(3, bp)
    bp = _step(2, 2, True)
    _gather(4, bp)

    def body(i, carry):
        j0 = 3 * i
        for b in range(3):
            bp = _step(j0 + b, b, True)
            _gather(j0 + b + 2, bp)
        return carry

    lax.fori_loop(1, (J - 2) // 3, body, 0, unroll=False)
    _step(J - 2, (J - 2) % 3, True)
    _step(J - 1, (J - 1) % 3, True)
    _drain(bufs[(J - 1) % 3], ss[(J - 1) % 3])
    plsc.subcore_barrier()
    pltpu.sync_copy(acc_sh.at[pl.ds(sid * RPT, RPT)],
                    out_hbm.at[cid].at[pl.ds(sid * RPT, RPT)])


def _tc_mid_body(acc_ref, ns_ref, nd_ref, b1_ref, w2_ref, wr_ref, t_ref):
    w = jnp.dot(w2_ref[...], wr_ref[...],
                precision=lax.Precision.HIGHEST)
    a = acc_ref[0] + acc_ref[1]
    h1 = jnp.maximum(nd_ref[...] * a + b1_ref[...][None, :], 0.0)
    t_ref[...] = jnp.dot(h1, w, precision=lax.Precision.HIGHEST) * ns_ref[...]


_tc_mid = pl.pallas_call(
    _tc_mid_body,
    grid=(G,),
    in_specs=[
        pl.BlockSpec((NC, RB, D), lambda i: (0, i, 0)),
        pl.BlockSpec((RB, 1), lambda i: (i, 0)),
        pl.BlockSpec((RB, 1), lambda i: (i, 0)),
        pl.BlockSpec((D,), lambda i: (0,)),
        pl.BlockSpec((D, D), lambda i: (0, 0)),
        pl.BlockSpec((D, 1), lambda i: (0, 0)),
    ],
    out_specs=[pl.BlockSpec((RB, 1), lambda i: (i, 0))],
    out_shape=[jax.ShapeDtypeStruct((N, 1), jnp.float32)],
)


@functools.partial(
    pl.kernel,
    out_type=jax.ShapeDtypeStruct((NC, N), jnp.float32),
    mesh=_mesh,
    compiler_params=_sc_params,
    scratch_types=[
        pltpu.VMEM((J, C), jnp.int32),
        pltpu.VMEM((J, C), jnp.int32),
        pltpu.VMEM((N,), jnp.float32),
        pltpu.VMEM((1, N), jnp.float32),
        pltpu.VMEM((1,), jnp.int32),
        pltpu.VMEM_SHARED((1, N), jnp.float32),
    ],
)
def _sc_scalar(edges_hbm, t_hbm, i0_hbm, out_hbm,
               sidx_v, didx_v, t_v, z_v, i0_v, zsum_sh):
    cid = lax.axis_index("c")
    sid = lax.axis_index("s")
    w = sid * NC + cid
    pltpu.sync_copy(edges_hbm.at[0, w], sidx_v)
    pltpu.sync_copy(edges_hbm.at[1, w], didx_v)
    pltpu.sync_copy(t_hbm, t_v)
    pltpu.sync_copy(i0_hbm, i0_v)
    z16 = jnp.zeros((16,), jnp.float32)

    def zbody(i, carry):
        z_v[0, pl.ds(i * 16, 16)] = z16
        return carry

    lax.fori_loop(0, N // 16, zbody, 0, unroll=False)

    @pl.when(sid == 0)
    def _():
        pltpu.sync_copy(z_v, zsum_sh)

    def body(j, carry):
        for k in range(C // 16):
            s16 = sidx_v[j, pl.ds(k * 16, 16)]
            vals = plsc.load_gather(t_v, [s16])
            d16 = didx_v[j, pl.ds(k * 16, 16)]
            plsc.addupdate_scatter(z_v.at[0], [d16], vals)
        return carry

    lax.fori_loop(0, J, body, 0, unroll=False)
    plsc.subcore_barrier()
    pltpu.sync_copy(z_v, zsum_sh.at[i0_v], add=True)
    plsc.subcore_barrier()

    @pl.when(sid == 0)
    def _():
        pltpu.sync_copy(zsum_sh.at[0], out_hbm.at[cid])


def _tc_final_body(zp_ref, t_ref, nd_ref, b2_ref, wr_ref, br_ref, out_ref):
    z = zp_ref[0] + zp_ref[1]
    c = jnp.sum(b2_ref[...] * wr_ref[...][:, 0]) + br_ref[0]
    out_ref[...] = nd_ref[...] * (z[:, None] + t_ref[...]) + c


_tc_final = pl.pallas_call(
    _tc_final_body,
    out_shape=[jax.ShapeDtypeStruct((N, 1), jnp.float32)],
)


def kernel(x, edge_index, W1, b1, W2, b2, Wr, br):
    edges = edge_index.reshape(2, NW, J, C)
    zrows = jnp.zeros((N, D), jnp.float32)
    i01 = jnp.arange(2, dtype=jnp.int32)
    i0 = jnp.zeros((1,), jnp.int32)

    hp = _sc_degrees(edges, i01)
    ns, nd = _tc_norms(hp)
    (y,) = _tc_prep(ns, x, W1)
    acc = _sc_gather_scatter(edges, y, zrows)
    (t,) = _tc_mid(acc, ns, nd, b1, W2, Wr)
    zp = _sc_scalar(edges, t[:, 0], i0)
    (out,) = _tc_final(zp, t, nd, b2, Wr, br)
    return out[:, 0]


# speedup vs baseline: 27.4762x; 1.0180x over previous
import functools

import jax
import jax.numpy as jnp
from jax import lax
from jax.experimental import pallas as pl
from jax.experimental.pallas import tpu as pltpu
from jax.experimental.pallas import tpu_sc as plsc

N = 10000
E = 320000
D = 128
NC = 2
NS = 16
NW = NC * NS
EP = E // NW
C = 80
J = EP // C
RPT = N // NS
RB = 2000
G = N // RB

_mesh = plsc.VectorSubcoreMesh(core_axis_name="c", subcore_axis_name="s")
_sc_params = pltpu.CompilerParams(
    use_tc_tiling_on_sc=False, needs_layout_passes=False)


@functools.partial(
    pl.kernel,
    out_type=jax.ShapeDtypeStruct((NC, 2, N), jnp.float32),
    mesh=_mesh,
    compiler_params=_sc_params,
    scratch_types=[
        pltpu.VMEM((J, C), jnp.int32),
        pltpu.VMEM((J, C), jnp.int32),
        pltpu.VMEM((2, N), jnp.float32),
        pltpu.VMEM((2,), jnp.int32),
        pltpu.VMEM_SHARED((2, N), jnp.float32),
    ],
)
def _sc_degrees(edges_hbm, i01_hbm, out_hbm, sidx_v, didx_v, deg_v, i01_v, hsum_sh):
    cid = lax.axis_index("c")
    sid = lax.axis_index("s")
    w = sid * NC + cid
    pltpu.sync_copy(edges_hbm.at[0, w], sidx_v)
    pltpu.sync_copy(edges_hbm.at[1, w], didx_v)
    pltpu.sync_copy(i01_hbm, i01_v)
    z16 = jnp.zeros((16,), jnp.float32)

    def zbody(i, carry):
        deg_v[0, pl.ds(i * 16, 16)] = z16
        deg_v[1, pl.ds(i * 16, 16)] = z16
        return carry

    lax.fori_loop(0, N // 16, zbody, 0, unroll=False)

    @pl.when(sid == 0)
    def _():
        pltpu.sync_copy(deg_v, hsum_sh)

    ones = jnp.ones((16,), jnp.float32)

    def body(j, carry):
        for k in range(C // 16):
            s16 = sidx_v[j, pl.ds(k * 16, 16)]
            plsc.addupdate_scatter(deg_v.at[0], [s16], ones)
            d16 = didx_v[j, pl.ds(k * 16, 16)]
            plsc.addupdate_scatter(deg_v.at[1], [d16], ones)
        return carry

    lax.fori_loop(0, J, body, 0, unroll=2)
    plsc.subcore_barrier()
    pltpu.sync_copy(deg_v, hsum_sh.at[i01_v], add=True)
    plsc.subcore_barrier()

    @pl.when(sid == 0)
    def _():
        pltpu.sync_copy(hsum_sh, out_hbm.at[cid])


def _tc_norms_body(hp_ref, ns_ref, nd_ref):
    deg = hp_ref[0] + hp_ref[1] + 1.0
    ns_ref[...] = lax.rsqrt(deg[0])[:, None]
    nd_ref[...] = lax.rsqrt(deg[1])[:, None]


_tc_norms = pl.pallas_call(
    _tc_norms_body,
    out_shape=[
        jax.ShapeDtypeStruct((N, 1), jnp.float32),
        jax.ShapeDtypeStruct((N, 1), jnp.float32),
    ],
)


def _tc_prep_body(ns_ref, x_ref, w1_ref, y_ref):
    y_ref[...] = jnp.dot(x_ref[...] * ns_ref[...], w1_ref[...],
                         precision=lax.Precision.HIGHEST)


_tc_prep = pl.pallas_call(
    _tc_prep_body,
    grid=(G,),
    in_specs=[
        pl.BlockSpec((RB, 1), lambda i: (i, 0)),
        pl.BlockSpec((RB, D), lambda i: (i, 0)),
        pl.BlockSpec((D, D), lambda i: (0, 0)),
    ],
    out_specs=[pl.BlockSpec((RB, D), lambda i: (i, 0))],
    out_shape=[jax.ShapeDtypeStruct((N, D), jnp.float32)],
)


@functools.partial(
    pl.kernel,
    out_type=jax.ShapeDtypeStruct((NC, N, D), jnp.float32),
    mesh=_mesh,
    compiler_params=_sc_params,
    scratch_types=[
        pltpu.VMEM((J, C), jnp.int32),
        pltpu.VMEM((J, C), jnp.int32),
        pltpu.VMEM((C, D), jnp.float32),
        pltpu.VMEM((C, D), jnp.float32),
        pltpu.VMEM((C, D), jnp.float32),
        pltpu.VMEM_SHARED((N, D), jnp.float32),
        pltpu.SemaphoreType.DMA,
        pltpu.SemaphoreType.DMA,
        pltpu.SemaphoreType.DMA,
        pltpu.SemaphoreType.DMA,
        pltpu.SemaphoreType.DMA,
        pltpu.SemaphoreType.DMA,
    ],
)
def _sc_gather_scatter(edges_hbm, y_hbm, out_hbm,
                       sidx_v, didx_v, rows0_v, rows1_v, rows2_v,
                       acc_sh, g0, g1, g2, s0, s1, s2):
    cid = lax.axis_index("c")
    sid = lax.axis_index("s")
    w = sid * NC + cid

    bufs = (rows0_v, rows1_v, rows2_v)
    gs = (g0, g1, g2)
    ss = (s0, s1, s2)

    def _drain(buf, sem):
        pltpu.make_async_copy(y_hbm.at[sidx_v.at[0]], buf, sem).wait()

    def _gather(j, b):
        pltpu.async_copy(y_hbm.at[sidx_v.at[j]], bufs[b], gs[b])

    def _step(j, b, drain_prev):
        _drain(bufs[b], gs[b])
        pltpu.async_copy(bufs[b], acc_sh.at[didx_v.at[j]], ss[b], add=True)
        bp = (b + 2) % 3
        if drain_prev:
            _drain(bufs[bp], ss[bp])
        return bp

    pltpu.sync_copy(edges_hbm.at[0, w], sidx_v)
    pltpu.sync_copy(edges_hbm.at[1, w], didx_v)
    _gather(0, 0)
    _gather(1, 1)

    @pl.when(cid == 0)
    def _():
        pltpu.sync_copy(y_hbm.at[pl.ds(sid * RPT, RPT)],
                        acc_sh.at[pl.ds(sid * RPT, RPT)])

    @pl.when(cid != 0)
    def _():
        z16 = jnp.zeros((16,), jnp.float32)

        def zb(i, carry):
            rows2_v[i, pl.ds(0, 16)] = z16
            rows2_v[i, pl.ds(16, 16)] = z16
            rows2_v[i, pl.ds(32, 16)] = z16
            rows2_v[i, pl.ds(48, 16)] = z16
            rows2_v[i, pl.ds(64, 16)] = z16
            rows2_v[i, pl.ds(80, 16)] = z16
            rows2_v[i, pl.ds(96, 16)] = z16
            rows2_v[i, pl.ds(112, 16)] = z16
            return carry

        lax.fori_loop(0, C, zb, 0, unroll=False)
        for q in range(7):
            pltpu.sync_copy(rows2_v,
                            acc_sh.at[pl.ds(sid * RPT + q * C, C)])
        pltpu.sync_copy(rows2_v.at[pl.ds(0, RPT - 7 * C)],
                        acc_sh.at[pl.ds(sid * RPT + 7 * C, RPT - 7 * C)])

    plsc.subcore_barrier()
    _step(0, 0, False)
    _gather(2, 2)
    bp = _step(1, 1, True)
    _gather(3, bp)
    bp = _step(2, 2, True)
    _gather(4, bp)

    def body(i, carry):
        j0 = 3 * i
        for b in range(3):
            bp = _step(j0 + b, b, True)
            _gather(j0 + b + 2, bp)
        return carry

    lax.fori_loop(1, (J - 2) // 3, body, 0, unroll=False)
    _step(J - 2, (J - 2) % 3, True)
    _step(J - 1, (J - 1) % 3, True)
    _drain(bufs[(J - 1) % 3], ss[(J - 1) % 3])
    plsc.subcore_barrier()
    pltpu.sync_copy(acc_sh.at[pl.ds(sid * RPT, RPT)],
                    out_hbm.at[cid].at[pl.ds(sid * RPT, RPT)])


def _tc_mid_body(acc_ref, ns_ref, nd_ref, b1_ref, w2_ref, wr_ref, t_ref):
    w = jnp.dot(w2_ref[...], wr_ref[...],
                precision=lax.Precision.HIGHEST)
    a = acc_ref[0] + acc_ref[1]
    h1 = jnp.maximum(nd_ref[...] * a + b1_ref[...][None, :], 0.0)
    t_ref[...] = jnp.dot(h1, w, precision=lax.Precision.HIGHEST) * ns_ref[...]


_tc_mid = pl.pallas_call(
    _tc_mid_body,
    grid=(G,),
    in_specs=[
        pl.BlockSpec((NC, RB, D), lambda i: (0, i, 0)),
        pl.BlockSpec((RB, 1), lambda i: (i, 0)),
        pl.BlockSpec((RB, 1), lambda i: (i, 0)),
        pl.BlockSpec((D,), lambda i: (0,)),
        pl.BlockSpec((D, D), lambda i: (0, 0)),
        pl.BlockSpec((D, 1), lambda i: (0, 0)),
    ],
    out_specs=[pl.BlockSpec((RB, 1), lambda i: (i, 0))],
    out_shape=[jax.ShapeDtypeStruct((N, 1), jnp.float32)],
)


@functools.partial(
    pl.kernel,
    out_type=jax.ShapeDtypeStruct((NC, N), jnp.float32),
    mesh=_mesh,
    compiler_params=_sc_params,
    scratch_types=[
        pltpu.VMEM((J, C), jnp.int32),
        pltpu.VMEM((J, C), jnp.int32),
        pltpu.VMEM((N,), jnp.float32),
        pltpu.VMEM((1, N), jnp.float32),
        pltpu.VMEM((1,), jnp.int32),
        pltpu.VMEM_SHARED((1, N), jnp.float32),
    ],
)
def _sc_scalar(edges_hbm, t_hbm, i0_hbm, out_hbm,
               sidx_v, didx_v, t_v, z_v, i0_v, zsum_sh):
    cid = lax.axis_index("c")
    sid = lax.axis_index("s")
    w = sid * NC + cid
    pltpu.sync_copy(edges_hbm.at[0, w], sidx_v)
    pltpu.sync_copy(edges_hbm.at[1, w], didx_v)
    pltpu.sync_copy(t_hbm, t_v)
    pltpu.sync_copy(i0_hbm, i0_v)
    z16 = jnp.zeros((16,), jnp.float32)

    def zbody(i, carry):
        z_v[0, pl.ds(i * 16, 16)] = z16
        return carry

    lax.fori_loop(0, N // 16, zbody, 0, unroll=False)

    @pl.when(sid == 0)
    def _():
        pltpu.sync_copy(z_v, zsum_sh)

    def body(j, carry):
        for k in range(C // 16):
            s16 = sidx_v[j, pl.ds(k * 16, 16)]
            vals = plsc.load_gather(t_v, [s16])
            d16 = didx_v[j, pl.ds(k * 16, 16)]
            plsc.addupdate_scatter(z_v.at[0], [d16], vals)
        return carry

    lax.fori_loop(0, J, body, 0, unroll=False)
    plsc.subcore_barrier()
    pltpu.sync_copy(z_v, zsum_sh.at[i0_v], add=True)
    plsc.subcore_barrier()

    @pl.when(sid == 0)
    def _():
        pltpu.sync_copy(zsum_sh.at[0], out_hbm.at[cid])


def _tc_final_body(zp_ref, t_ref, nd_ref, b2_ref, wr_ref, br_ref, out_ref):
    z = zp_ref[0] + zp_ref[1]
    c = jnp.sum(b2_ref[...] * wr_ref[...][:, 0]) + br_ref[0]
    out_ref[...] = nd_ref[...] * (z[:, None] + t_ref[...]) + c


_tc_final = pl.pallas_call(
    _tc_final_body,
    out_shape=[jax.ShapeDtypeStruct((N, 1), jnp.float32)],
)


def kernel(x, edge_index, W1, b1, W2, b2, Wr, br):
    edges = edge_index.reshape(2, NW, J, C)
    i01 = jnp.arange(2, dtype=jnp.int32)
    i0 = jnp.zeros((1,), jnp.int32)

    hp = _sc_degrees(edges, i01)
    ns, nd = _tc_norms(hp)
    (y,) = _tc_prep(ns, x, W1)
    acc = _sc_gather_scatter(edges, y)
    (t,) = _tc_mid(acc, ns, nd, b1, W2, Wr)
    zp = _sc_scalar(edges, t[:, 0], i0)
    (out,) = _tc_final(zp, t, nd, b2, Wr, br)
    return out[:, 0]
